# baseline (speedup 1.0000x reference)
import functools

import jax
import jax.numpy as jnp
from jax import lax
from jax.experimental import pallas as pl
from jax.experimental.pallas import tpu as pltpu
from jax.experimental.pallas import tpu_sc as plsc

DIM = 128
EDIM = 16
NN = 10000
NE = 320000

NC = 2
NS = 16
NW = NC * NS
CHUNK = 64
NCHT = NE // CHUNK
MAXCH = (NCHT + NW - 1) // NW
RPT = 624
RTAIL = NN - NS * RPT
SRCW = 3 * DIM
DSTW = 2 * DIM
PACK = DIM // EDIM
NER = NE // PACK

_F32 = jnp.float32


def _sigmoid(x):
    return 1.0 / (1.0 + jnp.exp(-x))


_BN = 2000


def _node_pre_body(h_ref, wn_ref, bn_ref, wy_ref, src_ref, dst_ref):
    xall = jnp.dot(h_ref[...], wn_ref[...], preferred_element_type=_F32)
    xall = xall + bn_ref[...]
    xs = xall[:, :DIM]
    xd = xall[:, DIM:2 * DIM]
    xm = xall[:, 2 * DIM:]
    y = jnp.dot(xm, wy_ref[...], preferred_element_type=_F32)
    pad = jnp.zeros((_BN, DIM - EDIM), _F32)
    src_ref[...] = jnp.concatenate([xs, xm, y[:, :EDIM], pad], axis=1)
    dst_ref[...] = jnp.concatenate([xd, y[:, EDIM:], pad], axis=1)


_node_pre = pl.pallas_call(
    _node_pre_body,
    grid=(NN // _BN,),
    in_specs=[
        pl.BlockSpec((_BN, DIM), lambda i: (i, 0)),
        pl.BlockSpec((DIM, 3 * DIM), lambda i: (0, 0)),
        pl.BlockSpec((1, 3 * DIM), lambda i: (0, 0)),
        pl.BlockSpec((DIM, 2 * EDIM), lambda i: (0, 0)),
    ],
    out_specs=[
        pl.BlockSpec((_BN, SRCW), lambda i: (i, 0)),
        pl.BlockSpec((_BN, DSTW), lambda i: (i, 0)),
    ],
    out_shape=[
        jax.ShapeDtypeStruct((NN, SRCW), _F32),
        jax.ShapeDtypeStruct((NN, DSTW), _F32),
    ],
)


_BE = 8000


def _eg_body(e_ref, w_ref, b_ref, o_ref):
    o_ref[...] = (jnp.dot(e_ref[...], w_ref[...], preferred_element_type=_F32)
                  + b_ref[...])


_eg_pre = pl.pallas_call(
    _eg_body,
    grid=(NE // _BE,),
    in_specs=[
        pl.BlockSpec((_BE, EDIM), lambda i: (i, 0)),
        pl.BlockSpec((EDIM, DIM), lambda i: (0, 0)),
        pl.BlockSpec((1, DIM), lambda i: (0, 0)),
    ],
    out_specs=pl.BlockSpec((_BE, DIM), lambda i: (i, 0)),
    out_shape=jax.ShapeDtypeStruct((NE, DIM), _F32),
)


_sc_mesh = plsc.VectorSubcoreMesh(core_axis_name="c", subcore_axis_name="s")


@functools.partial(
    pl.kernel,
    mesh=_sc_mesh,
    out_type=[
        jax.ShapeDtypeStruct((NC, NN, DIM), _F32),
        jax.ShapeDtypeStruct((NER, DIM), _F32),
    ],
    scratch_types=[
        pltpu.VMEM((CHUNK,), jnp.int32),
        pltpu.VMEM((CHUNK,), jnp.int32),
        pltpu.VMEM((CHUNK, SRCW), _F32),
        pltpu.VMEM((CHUNK, DSTW), _F32),
        pltpu.VMEM((CHUNK, DIM), _F32),
        pltpu.VMEM((CHUNK // PACK, DIM), _F32),
        pltpu.VMEM_SHARED((NN, DIM), _F32),
        pltpu.SemaphoreType.DMA,
        pltpu.SemaphoreType.DMA,
        pltpu.SemaphoreType.DMA,
    ],
)
def _sc_edge(src_tab, dst_tab, eg, src_idx, dst_idx, agg_out, s16_out,
             sidx, didx, sbuf, dbuf, egbuf, s16buf, aggsh,
             sem_s, sem_d, sem_e):
    cid = lax.axis_index("c")
    sid = lax.axis_index("s")
    wid = cid * NS + sid

    zero16 = jnp.zeros((16,), _F32)

    def _zrow(i, carry):
        for j in range(DIM // 16):
            egbuf[i, pl.ds(j * 16, 16)] = zero16
        return carry

    lax.fori_loop(0, CHUNK, _zrow, 0)
    rbase = sid * RPT
    for z in range(RPT // CHUNK):
        pltpu.sync_copy(egbuf, aggsh.at[pl.ds(rbase + z * CHUNK, CHUNK)])
    _ztail = RPT - (RPT // CHUNK) * CHUNK
    pltpu.sync_copy(egbuf.at[pl.ds(0, _ztail)],
                    aggsh.at[pl.ds(rbase + RPT - _ztail, _ztail)])

    @pl.when(sid == 0)
    def _zero_tail():
        pltpu.sync_copy(egbuf.at[pl.ds(0, RTAIL)],
                        aggsh.at[pl.ds(NS * RPT, RTAIL)])

    plsc.subcore_barrier()

    def _chunk(t, carry):
        ci = wid + t * NW

        @pl.when(ci < NCHT)
        def _():
            base = ci * CHUNK
            pltpu.sync_copy(src_idx.at[pl.ds(base, CHUNK)], sidx)
            pltpu.sync_copy(dst_idx.at[pl.ds(base, CHUNK)], didx)
            cp_s = pltpu.async_copy(src_tab.at[sidx], sbuf, sem_s)
            cp_d = pltpu.async_copy(dst_tab.at[didx], dbuf, sem_d)
            cp_e = pltpu.async_copy(eg.at[pl.ds(base, CHUNK)], egbuf, sem_e)
            cp_s.wait()
            cp_d.wait()
            cp_e.wait()

            @plsc.parallel_loop(0, CHUNK, unroll=4)
            def _row(i):
                for j in range(DIM // 16):
                    xs = sbuf[i, pl.ds(j * 16, 16)]
                    xd = dbuf[i, pl.ds(j * 16, 16)]
                    ge = egbuf[i, pl.ds(j * 16, 16)]
                    xm = sbuf[i, pl.ds(DIM + j * 16, 16)]
                    egbuf[i, pl.ds(j * 16, 16)] = (
                        xm / (1.0 + jnp.exp(-(xs + xd + ge))))
                ys = sbuf[i, pl.ds(2 * DIM, EDIM)]
                yd = dbuf[i, pl.ds(DIM, EDIM)]
                s16buf[i // PACK, pl.ds((i % PACK) * EDIM, EDIM)] = ys + yd

            pltpu.sync_copy(egbuf, aggsh.at[didx], add=True)
            pltpu.sync_copy(s16buf, s16_out.at[pl.ds(ci * (CHUNK // PACK),
                                                     CHUNK // PACK)])

        return carry

    lax.fori_loop(0, MAXCH, _chunk, 0)

    plsc.subcore_barrier()
    pltpu.sync_copy(aggsh.at[pl.ds(rbase, RPT)],
                    agg_out.at[cid, pl.ds(rbase, RPT)])

    @pl.when(sid == 0)
    def _write_tail():
        pltpu.sync_copy(aggsh.at[pl.ds(NS * RPT, RTAIL)],
                        agg_out.at[cid, pl.ds(NS * RPT, RTAIL)])


def _node_post_body(h_ref, a0_ref, a1_ref, w1h_ref, w1a_ref, b1_ref,
                    w2_ref, b2_ref, g_ref, bn_ref, o_ref):
    h = h_ref[...]
    agg = a0_ref[0] + a1_ref[0]
    t = (jnp.dot(h, w1h_ref[...], preferred_element_type=_F32)
         + jnp.dot(agg, w1a_ref[...], preferred_element_type=_F32)
         + b1_ref[...])
    t = t * _sigmoid(t)
    nu = jnp.dot(t, w2_ref[...], preferred_element_type=_F32) + b2_ref[...]
    x = h + nu
    mu = jnp.mean(x, axis=1, keepdims=True)
    d = x - mu
    var = jnp.mean(d * d, axis=1, keepdims=True)
    o_ref[...] = d * lax.rsqrt(var + 1e-5) * g_ref[...] + bn_ref[...]


_node_post = pl.pallas_call(
    _node_post_body,
    grid=(NN // _BN,),
    in_specs=[
        pl.BlockSpec((_BN, DIM), lambda i: (i, 0)),
        pl.BlockSpec((1, _BN, DIM), lambda i: (0, i, 0)),
        pl.BlockSpec((1, _BN, DIM), lambda i: (1, i, 0)),
        pl.BlockSpec((DIM, DIM), lambda i: (0, 0)),
        pl.BlockSpec((DIM, DIM), lambda i: (0, 0)),
        pl.BlockSpec((1, DIM), lambda i: (0, 0)),
        pl.BlockSpec((DIM, DIM), lambda i: (0, 0)),
        pl.BlockSpec((1, DIM), lambda i: (0, 0)),
        pl.BlockSpec((1, DIM), lambda i: (0, 0)),
        pl.BlockSpec((1, DIM), lambda i: (0, 0)),
    ],
    out_specs=pl.BlockSpec((_BN, DIM), lambda i: (i, 0)),
    out_shape=jax.ShapeDtypeStruct((NN, DIM), _F32),
)


_BEP = 8000


def _edge_post_body(e_ref, s_ref, w1_ref, b1_ref, w2_ref, b2_ref,
                    gm_ref, g_ref, bn_ref, o_ref):
    eb = e_ref[...]
    t1 = (jnp.dot(eb, w1_ref[...], preferred_element_type=_F32)
          + s_ref[...] + b1_ref[...])
    t = t1 * _sigmoid(t1)
    eu = jnp.dot(t, w2_ref[...], preferred_element_type=_F32) + b2_ref[...]
    x = eb + eu
    mu = jnp.dot(x, gm_ref[...], preferred_element_type=_F32)
    d = x - mu
    var = jnp.dot(d * d, gm_ref[...], preferred_element_type=_F32)
    o_ref[...] = d * lax.rsqrt(var + 1e-5) * g_ref[...] + bn_ref[...]


_edge_post = pl.pallas_call(
    _edge_post_body,
    grid=(NER // _BEP,),
    in_specs=[
        pl.BlockSpec((_BEP, DIM), lambda i: (i, 0)),
        pl.BlockSpec((_BEP, DIM), lambda i: (i, 0)),
        pl.BlockSpec((DIM, DIM), lambda i: (0, 0)),
        pl.BlockSpec((1, DIM), lambda i: (0, 0)),
        pl.BlockSpec((DIM, DIM), lambda i: (0, 0)),
        pl.BlockSpec((1, DIM), lambda i: (0, 0)),
        pl.BlockSpec((DIM, DIM), lambda i: (0, 0)),
        pl.BlockSpec((1, DIM), lambda i: (0, 0)),
        pl.BlockSpec((1, DIM), lambda i: (0, 0)),
    ],
    out_specs=pl.BlockSpec((_BEP, DIM), lambda i: (i, 0)),
    out_shape=jax.ShapeDtypeStruct((NER, DIM), _F32),
)


def kernel(h, e, edge_index, params):
    p = params
    src = edge_index[0].astype(jnp.int32)
    dst = edge_index[1].astype(jnp.int32)

    wn = jnp.concatenate([p['src_W'].T, p['dst_W'].T, p['msg_W'].T], axis=1)
    bn = jnp.concatenate([p['src_b'], p['dst_b'], p['msg_b']])[None, :]
    wy = jnp.concatenate([p['eu_W1'][:, EDIM:EDIM + DIM].T,
                          p['eu_W1'][:, EDIM + DIM:].T], axis=1)
    eye8 = jnp.eye(PACK, dtype=_F32)
    w1bd = jnp.kron(eye8, p['eu_W1'][:, :EDIM].T)
    w2bd = jnp.kron(eye8, p['eu_W2'].T)
    gmat = jnp.kron(eye8, jnp.full((EDIM, EDIM), 1.0 / EDIM, _F32))
    b1t = jnp.tile(p['eu_b1'], PACK)[None, :]
    b2t = jnp.tile(p['eu_b2'], PACK)[None, :]
    egt = jnp.tile(p['en_g'], PACK)[None, :]
    ebt = jnp.tile(p['en_b'], PACK)[None, :]

    src_tab, dst_tab = _node_pre(h, wn, bn, wy)
    eg = _eg_pre(e, p['eg_W'].T, p['eg_b'][None, :])
    agg2, s16 = _sc_edge(src_tab, dst_tab, eg, src, dst)

    h_new = _node_post(h, agg2, agg2,
                       p['nu_W1'][:, :DIM].T, p['nu_W1'][:, DIM:].T,
                       p['nu_b1'][None, :], p['nu_W2'].T,
                       p['nu_b2'][None, :], p['nn_g'][None, :],
                       p['nn_b'][None, :])
    e_new = _edge_post(e.reshape(NER, DIM), s16,
                       w1bd, b1t, w2bd, b2t, gmat, egt, ebt)
    return (h_new, e_new.reshape(NE, EDIM))


# speedup vs baseline: 1.1304x; 1.1304x over previous
import functools

import jax
import jax.numpy as jnp
from jax import lax
from jax.experimental import pallas as pl
from jax.experimental.pallas import tpu as pltpu
from jax.experimental.pallas import tpu_sc as plsc

DIM = 128
HDIM = DIM // 2
EDIM = 16
NN = 10000
NE = 320000

NC = 2
NS = 16
NW = NC * NS
CHUNK = 64
NCHT = NE // CHUNK
MAXCH = (NCHT + NW - 1) // NW
RPT = 624
RTAIL = NN - NS * RPT
SRCW = 2 * DIM
DSTW = DIM
EGW = HDIM
PACK = DIM // EDIM
NER = NE // PACK

_F32 = jnp.float32
_U32 = jnp.uint32


def _sigmoid(x):
    return 1.0 / (1.0 + jnp.exp(-x))


def _pk(lo, hi):
    ulo = lax.bitcast_convert_type(lo, _U32)
    uhi = lax.bitcast_convert_type(hi, _U32)
    r = _U32(0x8000)
    return (((ulo + r) >> _U32(16)) | ((uhi + r) & _U32(0xFFFF0000)))


_BN = 2000


def _node_pre_body(h_ref, wls_ref, bls_ref, whs_ref, bhs_ref,
                   wld_ref, bld_ref, whd_ref, bhd_ref, src_ref, dst_ref):
    h = h_ref[...]
    lo_s = jnp.dot(h, wls_ref[...], preferred_element_type=_F32) + bls_ref[...]
    hi_s = jnp.dot(h, whs_ref[...], preferred_element_type=_F32) + bhs_ref[...]
    lo_d = jnp.dot(h, wld_ref[...], preferred_element_type=_F32) + bld_ref[...]
    hi_d = jnp.dot(h, whd_ref[...], preferred_element_type=_F32) + bhd_ref[...]
    src_ref[...] = _pk(lo_s, hi_s)
    dst_ref[...] = _pk(lo_d, hi_d)


_node_pre = pl.pallas_call(
    _node_pre_body,
    grid=(NN // _BN,),
    in_specs=[
        pl.BlockSpec((_BN, DIM), lambda i: (i, 0)),
        pl.BlockSpec((DIM, SRCW), lambda i: (0, 0)),
        pl.BlockSpec((1, SRCW), lambda i: (0, 0)),
        pl.BlockSpec((DIM, SRCW), lambda i: (0, 0)),
        pl.BlockSpec((1, SRCW), lambda i: (0, 0)),
        pl.BlockSpec((DIM, DSTW), lambda i: (0, 0)),
        pl.BlockSpec((1, DSTW), lambda i: (0, 0)),
        pl.BlockSpec((DIM, DSTW), lambda i: (0, 0)),
        pl.BlockSpec((1, DSTW), lambda i: (0, 0)),
    ],
    out_specs=[
        pl.BlockSpec((_BN, SRCW), lambda i: (i, 0)),
        pl.BlockSpec((_BN, DSTW), lambda i: (i, 0)),
    ],
    out_shape=[
        jax.ShapeDtypeStruct((NN, SRCW), _U32),
        jax.ShapeDtypeStruct((NN, DSTW), _U32),
    ],
)


_BE = 8000


def _eg_body(e_ref, wl_ref, bl_ref, wh_ref, bh_ref, o_ref):
    e = e_ref[...]
    lo = jnp.dot(e, wl_ref[...], preferred_element_type=_F32) + bl_ref[...]
    hi = jnp.dot(e, wh_ref[...], preferred_element_type=_F32) + bh_ref[...]
    o_ref[...] = _pk(lo, hi)


_eg_pre = pl.pallas_call(
    _eg_body,
    grid=(NE // _BE,),
    in_specs=[
        pl.BlockSpec((_BE, EDIM), lambda i: (i, 0)),
        pl.BlockSpec((EDIM, EGW), lambda i: (0, 0)),
        pl.BlockSpec((1, EGW), lambda i: (0, 0)),
        pl.BlockSpec((EDIM, EGW), lambda i: (0, 0)),
        pl.BlockSpec((1, EGW), lambda i: (0, 0)),
    ],
    out_specs=pl.BlockSpec((_BE, EGW), lambda i: (i, 0)),
    out_shape=jax.ShapeDtypeStruct((NE, EGW), _U32),
)


_sc_mesh = plsc.VectorSubcoreMesh(core_axis_name="c", subcore_axis_name="s")


@functools.partial(
    pl.kernel,
    mesh=_sc_mesh,
    compiler_params=pltpu.CompilerParams(needs_layout_passes=False),
    out_type=[
        jax.ShapeDtypeStruct((NC, NN, DIM), _F32),
        jax.ShapeDtypeStruct((NER, DIM), _F32),
    ],
    scratch_types=[
        pltpu.VMEM((CHUNK,), jnp.int32),
        pltpu.VMEM((CHUNK,), jnp.int32),
        pltpu.VMEM((CHUNK, SRCW), _U32),
        pltpu.VMEM((CHUNK, DSTW), _U32),
        pltpu.VMEM((CHUNK, EGW), _U32),
        pltpu.VMEM((CHUNK, DIM), _F32),
        pltpu.VMEM((CHUNK // PACK, DIM), _F32),
        pltpu.VMEM_SHARED((NN, DIM), _F32),
        pltpu.SemaphoreType.DMA,
        pltpu.SemaphoreType.DMA,
        pltpu.SemaphoreType.DMA,
    ],
)
def _sc_edge(src_tab, dst_tab, eg, src_idx, dst_idx, agg_out, s16_out,
             sidx, didx, sbuf, dbuf, egbuf, mbuf, s16buf, aggsh,
             sem_s, sem_d, sem_e):
    cid = lax.axis_index("c")
    sid = lax.axis_index("s")
    wid = cid * NS + sid

    def _unlo(w):
        return plsc.bitcast(w << _U32(16), _F32)

    def _unhi(w):
        return plsc.bitcast(w & _U32(0xFFFF0000), _F32)

    zero16 = jnp.zeros((16,), _F32)

    @plsc.parallel_loop(0, CHUNK, unroll=4)
    def _zrow(i):
        for j in range(DIM // 16):
            mbuf[i, pl.ds(j * 16, 16)] = zero16

    rbase = sid * RPT
    for z in range(RPT // CHUNK):
        pltpu.sync_copy(mbuf, aggsh.at[pl.ds(rbase + z * CHUNK, CHUNK)])
    _ztail = RPT - (RPT // CHUNK) * CHUNK
    pltpu.sync_copy(mbuf.at[pl.ds(0, _ztail)],
                    aggsh.at[pl.ds(rbase + RPT - _ztail, _ztail)])

    @pl.when(sid == 0)
    def _zero_tail():
        pltpu.sync_copy(mbuf.at[pl.ds(0, RTAIL)],
                        aggsh.at[pl.ds(NS * RPT, RTAIL)])

    plsc.subcore_barrier()

    def _chunk(t, carry):
        ci = wid + t * NW

        @pl.when(ci < NCHT)
        def _():
            base = ci * CHUNK
            pltpu.sync_copy(src_idx.at[pl.ds(base, CHUNK)], sidx)
            pltpu.sync_copy(dst_idx.at[pl.ds(base, CHUNK)], didx)
            cp_s = pltpu.async_copy(src_tab.at[sidx], sbuf, sem_s)
            cp_d = pltpu.async_copy(dst_tab.at[didx], dbuf, sem_d)
            cp_e = pltpu.async_copy(eg.at[pl.ds(base, CHUNK)], egbuf, sem_e)
            cp_s.wait()
            cp_d.wait()
            cp_e.wait()

            @plsc.parallel_loop(0, CHUNK, unroll=4)
            def _row(i):
                for jj in range(4):
                    ws_a = sbuf[i, pl.ds(jj * 16, 16)]
                    ws_b = sbuf[i, pl.ds((jj + 4) * 16, 16)]
                    wd = dbuf[i, pl.ds(jj * 16, 16)]
                    we = egbuf[i, pl.ds(jj * 16, 16)]
                    za = _unlo(ws_a) + _unlo(wd) + _unlo(we)
                    zb = _unlo(ws_b) + _unhi(wd) + _unhi(we)
                    mbuf[i, pl.ds(jj * 16, 16)] = (
                        _unhi(ws_a) / (1.0 + jnp.exp(-za)))
                    mbuf[i, pl.ds((jj + 4) * 16, 16)] = (
                        _unhi(ws_b) / (1.0 + jnp.exp(-zb)))
                wy = sbuf[i, pl.ds(DIM, 16)]
                wyd = dbuf[i, pl.ds(HDIM, 16)]
                s16buf[i // PACK, pl.ds((i % PACK) * EDIM, EDIM)] = (
                    _unlo(wy) + _unlo(wyd))

            pltpu.sync_copy(mbuf, aggsh.at[didx], add=True)
            pltpu.sync_copy(s16buf, s16_out.at[pl.ds(ci * (CHUNK // PACK),
                                                     CHUNK // PACK)])

        return carry

    lax.fori_loop(0, MAXCH, _chunk, 0)

    plsc.subcore_barrier()
    pltpu.sync_copy(aggsh.at[pl.ds(rbase, RPT)],
                    agg_out.at[cid, pl.ds(rbase, RPT)])

    @pl.when(sid == 0)
    def _write_tail():
        pltpu.sync_copy(aggsh.at[pl.ds(NS * RPT, RTAIL)],
                        agg_out.at[cid, pl.ds(NS * RPT, RTAIL)])


def _node_post_body(h_ref, a0_ref, a1_ref, w1h_ref, w1a_ref, b1_ref,
                    w2_ref, b2_ref, g_ref, bn_ref, o_ref):
    h = h_ref[...]
    agg = a0_ref[0] + a1_ref[0]
    t = (jnp.dot(h, w1h_ref[...], preferred_element_type=_F32)
         + jnp.dot(agg, w1a_ref[...], preferred_element_type=_F32)
         + b1_ref[...])
    t = t * _sigmoid(t)
    nu = jnp.dot(t, w2_ref[...], preferred_element_type=_F32) + b2_ref[...]
    x = h + nu
    mu = jnp.mean(x, axis=1, keepdims=True)
    d = x - mu
    var = jnp.mean(d * d, axis=1, keepdims=True)
    o_ref[...] = d * lax.rsqrt(var + 1e-5) * g_ref[...] + bn_ref[...]


_node_post = pl.pallas_call(
    _node_post_body,
    grid=(NN // _BN,),
    in_specs=[
        pl.BlockSpec((_BN, DIM), lambda i: (i, 0)),
        pl.BlockSpec((1, _BN, DIM), lambda i: (0, i, 0)),
        pl.BlockSpec((1, _BN, DIM), lambda i: (1, i, 0)),
        pl.BlockSpec((DIM, DIM), lambda i: (0, 0)),
        pl.BlockSpec((DIM, DIM), lambda i: (0, 0)),
        pl.BlockSpec((1, DIM), lambda i: (0, 0)),
        pl.BlockSpec((DIM, DIM), lambda i: (0, 0)),
        pl.BlockSpec((1, DIM), lambda i: (0, 0)),
        pl.BlockSpec((1, DIM), lambda i: (0, 0)),
        pl.BlockSpec((1, DIM), lambda i: (0, 0)),
    ],
    out_specs=pl.BlockSpec((_BN, DIM), lambda i: (i, 0)),
    out_shape=jax.ShapeDtypeStruct((NN, DIM), _F32),
)


_BEP = 8000


def _edge_post_body(e_ref, s_ref, w1_ref, b1_ref, w2_ref, b2_ref,
                    gm_ref, g_ref, bn_ref, o_ref):
    eb = e_ref[...]
    t1 = (jnp.dot(eb, w1_ref[...], preferred_element_type=_F32)
          + s_ref[...] + b1_ref[...])
    t = t1 * _sigmoid(t1)
    eu = jnp.dot(t, w2_ref[...], preferred_element_type=_F32) + b2_ref[...]
    x = eb + eu
    mu = jnp.dot(x, gm_ref[...], preferred_element_type=_F32)
    d = x - mu
    var = jnp.dot(d * d, gm_ref[...], preferred_element_type=_F32)
    o_ref[...] = d * lax.rsqrt(var + 1e-5) * g_ref[...] + bn_ref[...]


_edge_post = pl.pallas_call(
    _edge_post_body,
    grid=(NER // _BEP,),
    in_specs=[
        pl.BlockSpec((_BEP, DIM), lambda i: (i, 0)),
        pl.BlockSpec((_BEP, DIM), lambda i: (i, 0)),
        pl.BlockSpec((DIM, DIM), lambda i: (0, 0)),
        pl.BlockSpec((1, DIM), lambda i: (0, 0)),
        pl.BlockSpec((DIM, DIM), lambda i: (0, 0)),
        pl.BlockSpec((1, DIM), lambda i: (0, 0)),
        pl.BlockSpec((DIM, DIM), lambda i: (0, 0)),
        pl.BlockSpec((1, DIM), lambda i: (0, 0)),
        pl.BlockSpec((1, DIM), lambda i: (0, 0)),
    ],
    out_specs=pl.BlockSpec((_BEP, DIM), lambda i: (i, 0)),
    out_shape=jax.ShapeDtypeStruct((NER, DIM), _F32),
)


def kernel(h, e, edge_index, params):
    p = params
    src = edge_index[0].astype(jnp.int32)
    dst = edge_index[1].astype(jnp.int32)

    wys = p['eu_W1'][:, EDIM:EDIM + DIM].T
    wyd = p['eu_W1'][:, EDIM + DIM:].T
    z48 = jnp.zeros((DIM, SRCW - DIM - EDIM), _F32)
    wls = jnp.concatenate([p['src_W'].T, p['msg_W'].T @ wys, z48], axis=1)
    bls = jnp.concatenate([p['src_b'], p['msg_b'] @ wys,
                           jnp.zeros((SRCW - DIM - EDIM,), _F32)])[None, :]
    whs = jnp.concatenate([p['msg_W'].T, jnp.zeros((DIM, SRCW - DIM), _F32)],
                          axis=1)
    bhs = jnp.concatenate([p['msg_b'],
                           jnp.zeros((SRCW - DIM,), _F32)])[None, :]
    zd = jnp.zeros((DIM, DSTW - HDIM - EDIM), _F32)
    wld = jnp.concatenate([p['dst_W'].T[:, :HDIM], p['msg_W'].T @ wyd, zd],
                          axis=1)
    bld = jnp.concatenate([p['dst_b'][:HDIM], p['msg_b'] @ wyd,
                           jnp.zeros((DSTW - HDIM - EDIM,), _F32)])[None, :]
    whd = jnp.concatenate([p['dst_W'].T[:, HDIM:],
                           jnp.zeros((DIM, DSTW - HDIM), _F32)], axis=1)
    bhd = jnp.concatenate([p['dst_b'][HDIM:],
                           jnp.zeros((DSTW - HDIM,), _F32)])[None, :]
    wle = p['eg_W'].T[:, :HDIM]
    ble = p['eg_b'][:HDIM][None, :]
    whe = p['eg_W'].T[:, HDIM:]
    bhe = p['eg_b'][HDIM:][None, :]

    eye8 = jnp.eye(PACK, dtype=_F32)
    w1bd = jnp.kron(eye8, p['eu_W1'][:, :EDIM].T)
    w2bd = jnp.kron(eye8, p['eu_W2'].T)
    gmat = jnp.kron(eye8, jnp.full((EDIM, EDIM), 1.0 / EDIM, _F32))
    b1t = jnp.tile(p['eu_b1'], PACK)[None, :]
    b2t = jnp.tile(p['eu_b2'], PACK)[None, :]
    egt = jnp.tile(p['en_g'], PACK)[None, :]
    ebt = jnp.tile(p['en_b'], PACK)[None, :]

    src_tab, dst_tab = _node_pre(h, wls, bls, whs, bhs, wld, bld, whd, bhd)
    eg = _eg_pre(e, wle, ble, whe, bhe)
    agg2, s16 = _sc_edge(src_tab, dst_tab, eg, src, dst)

    h_new = _node_post(h, agg2, agg2,
                       p['nu_W1'][:, :DIM].T, p['nu_W1'][:, DIM:].T,
                       p['nu_b1'][None, :], p['nu_W2'].T,
                       p['nu_b2'][None, :], p['nn_g'][None, :],
                       p['nn_b'][None, :])
    e_new = _edge_post(e.reshape(NER, DIM), s16,
                       w1bd, b1t, w2bd, b2t, gmat, egt, ebt)
    return (h_new, e_new.reshape(NE, EDIM))


# speedup vs baseline: 1.5544x; 1.3751x over previous
import functools

import jax
import jax.numpy as jnp
from jax import lax
from jax.experimental import pallas as pl
from jax.experimental.pallas import tpu as pltpu
from jax.experimental.pallas import tpu_sc as plsc

DIM = 128
HDIM = DIM // 2
EDIM = 16
NN = 10000
NE = 320000

NC = 2
NS = 16
NW = NC * NS
CHUNK = 64
NCHT = NE // CHUNK
MAXCH = (NCHT + NW - 1) // NW
RPT = 624
RTAIL = NN - NS * RPT
SRCW = 2 * DIM
DSTW = DIM
EGW = HDIM
PACK = DIM // EDIM
NER = NE // PACK

_F32 = jnp.float32
_U32 = jnp.uint32


def _sigmoid(x):
    return 1.0 / (1.0 + jnp.exp(-x))


def _pk(lo, hi):
    ulo = lax.bitcast_convert_type(lo, _U32)
    uhi = lax.bitcast_convert_type(hi, _U32)
    r = _U32(0x8000)
    return (((ulo + r) >> _U32(16)) | ((uhi + r) & _U32(0xFFFF0000)))


_BN = 2000


def _node_pre_body(h_ref, wls_ref, bls_ref, whs_ref, bhs_ref,
                   wld_ref, bld_ref, whd_ref, bhd_ref, src_ref, dst_ref):
    h = h_ref[...]
    lo_s = jnp.dot(h, wls_ref[...], preferred_element_type=_F32) + bls_ref[...]
    hi_s = jnp.dot(h, whs_ref[...], preferred_element_type=_F32) + bhs_ref[...]
    lo_d = jnp.dot(h, wld_ref[...], preferred_element_type=_F32) + bld_ref[...]
    hi_d = jnp.dot(h, whd_ref[...], preferred_element_type=_F32) + bhd_ref[...]
    src_ref[...] = _pk(lo_s, hi_s)
    dst_ref[...] = _pk(lo_d, hi_d)


_node_pre = pl.pallas_call(
    _node_pre_body,
    grid=(NN // _BN,),
    in_specs=[
        pl.BlockSpec((_BN, DIM), lambda i: (i, 0)),
        pl.BlockSpec((DIM, SRCW), lambda i: (0, 0)),
        pl.BlockSpec((1, SRCW), lambda i: (0, 0)),
        pl.BlockSpec((DIM, SRCW), lambda i: (0, 0)),
        pl.BlockSpec((1, SRCW), lambda i: (0, 0)),
        pl.BlockSpec((DIM, DSTW), lambda i: (0, 0)),
        pl.BlockSpec((1, DSTW), lambda i: (0, 0)),
        pl.BlockSpec((DIM, DSTW), lambda i: (0, 0)),
        pl.BlockSpec((1, DSTW), lambda i: (0, 0)),
    ],
    out_specs=[
        pl.BlockSpec((_BN, SRCW), lambda i: (i, 0)),
        pl.BlockSpec((_BN, DSTW), lambda i: (i, 0)),
    ],
    out_shape=[
        jax.ShapeDtypeStruct((NN, SRCW), _U32),
        jax.ShapeDtypeStruct((NN, DSTW), _U32),
    ],
)


_BE = 8000


def _eg_body(e_ref, wl_ref, bl_ref, wh_ref, bh_ref, o_ref):
    e = e_ref[...]
    lo = jnp.dot(e, wl_ref[...], preferred_element_type=_F32) + bl_ref[...]
    hi = jnp.dot(e, wh_ref[...], preferred_element_type=_F32) + bh_ref[...]
    o_ref[...] = _pk(lo, hi)


_eg_pre = pl.pallas_call(
    _eg_body,
    grid=(NE // _BE,),
    in_specs=[
        pl.BlockSpec((_BE, EDIM), lambda i: (i, 0)),
        pl.BlockSpec((EDIM, EGW), lambda i: (0, 0)),
        pl.BlockSpec((1, EGW), lambda i: (0, 0)),
        pl.BlockSpec((EDIM, EGW), lambda i: (0, 0)),
        pl.BlockSpec((1, EGW), lambda i: (0, 0)),
    ],
    out_specs=pl.BlockSpec((_BE, EGW), lambda i: (i, 0)),
    out_shape=jax.ShapeDtypeStruct((NE, EGW), _U32),
)


_sc_mesh = plsc.VectorSubcoreMesh(core_axis_name="c", subcore_axis_name="s")


HALF = CHUNK // 2
RING = 8
RINGE = RING * CHUNK
IDXPAD = 2 * RINGE
NCHW0 = MAXCH


@functools.partial(
    pl.kernel,
    mesh=_sc_mesh,
    compiler_params=pltpu.CompilerParams(needs_layout_passes=False),
    out_type=[
        jax.ShapeDtypeStruct((NC, NN, DIM), _F32),
        jax.ShapeDtypeStruct((NER, DIM), _F32),
    ],
    scratch_types=[
        pltpu.VMEM((2 * RINGE,), jnp.int32),
        pltpu.VMEM((2 * RINGE,), jnp.int32),
        pltpu.VMEM((2, HALF), jnp.int32),
        pltpu.VMEM((2, HALF), jnp.int32),
        pltpu.VMEM((2, HALF, SRCW), _U32),
        pltpu.VMEM((2, HALF, DSTW), _U32),
        pltpu.VMEM((2, HALF, EGW), _U32),
        pltpu.VMEM((2, HALF, DIM), _F32),
        pltpu.VMEM((CHUNK // PACK, DIM), _F32),
        pltpu.VMEM_SHARED((NN, DIM), _F32),
        pltpu.SemaphoreType.DMA,
        pltpu.SemaphoreType.DMA,
        pltpu.SemaphoreType.DMA,
        pltpu.SemaphoreType.DMA,
        pltpu.SemaphoreType.DMA,
        pltpu.SemaphoreType.DMA,
    ],
)
def _sc_edge(src_tab, dst_tab, eg, src_idx, dst_idx, agg_out, s16_out,
             sring, dring, sidx2, didx2, sbuf2, dbuf2, egbuf2, mbuf2,
             s16buf, aggsh, sem_s0, sem_s1, sem_d0, sem_d1, sem_e0, sem_e1):
    cid = lax.axis_index("c")
    sid = lax.axis_index("s")
    wid = cid * NS + sid

    sems = ((sem_s0, sem_d0, sem_e0), (sem_s1, sem_d1, sem_e1))

    def _unlo(w):
        return plsc.bitcast(w << _U32(16), _F32)

    def _unhi(w):
        return plsc.bitcast(w & _U32(0xFFFF0000), _F32)

    zero16 = jnp.zeros((16,), _F32)

    @plsc.parallel_loop(0, HALF, unroll=4)
    def _zrow(i):
        for j in range(DIM // 16):
            mbuf2[0, i, pl.ds(j * 16, 16)] = zero16

    rbase = sid * RPT
    for z in range(RPT // HALF):
        pltpu.sync_copy(mbuf2.at[0], aggsh.at[pl.ds(rbase + z * HALF, HALF)])
    _ztail = RPT - (RPT // HALF) * HALF
    pltpu.sync_copy(mbuf2.at[0, pl.ds(0, _ztail)],
                    aggsh.at[pl.ds(rbase + RPT - _ztail, _ztail)])

    @pl.when(sid == 0)
    def _zero_tail():
        pltpu.sync_copy(mbuf2.at[0, pl.ds(0, RTAIL)],
                        aggsh.at[pl.ds(NS * RPT, RTAIL)])

    plsc.subcore_barrier()

    start = wid * (MAXCH - 1) + jnp.minimum(wid, NCHT - NW * (MAXCH - 1))
    nch = jnp.where(wid < NCHT - NW * (MAXCH - 1), MAXCH, MAXCH - 1)

    def _fill(p, tv, hv):
        pos = (((tv // RING) % 2) * RING + tv % RING) * CHUNK + hv * HALF
        for kk in range(HALF // 16):
            sidx2[p, pl.ds(kk * 16, 16)] = sring[pl.ds(pos + kk * 16, 16)]
            didx2[p, pl.ds(kk * 16, 16)] = dring[pl.ds(pos + kk * 16, 16)]

    def _issue(p, ci, hv):
        eb = ci * CHUNK + hv * HALF
        pltpu.async_copy(src_tab.at[sidx2.at[p]], sbuf2.at[p], sems[p][0])
        pltpu.async_copy(dst_tab.at[didx2.at[p]], dbuf2.at[p], sems[p][1])
        pltpu.async_copy(eg.at[pl.ds(eb, HALF)], egbuf2.at[p], sems[p][2])

    def _wait(p):
        pltpu.make_async_copy(src_tab.at[sidx2.at[p]], sbuf2.at[p],
                              sems[p][0]).wait()
        pltpu.make_async_copy(dst_tab.at[didx2.at[p]], dbuf2.at[p],
                              sems[p][1]).wait()
        pltpu.make_async_copy(eg.at[pl.ds(0, HALF)], egbuf2.at[p],
                              sems[p][2]).wait()

    def _compute(p):
        @plsc.parallel_loop(0, HALF, unroll=4)
        def _row(i):
            for jj in range(4):
                ws_a = sbuf2[p, i, pl.ds(jj * 16, 16)]
                ws_b = sbuf2[p, i, pl.ds((jj + 4) * 16, 16)]
                wd = dbuf2[p, i, pl.ds(jj * 16, 16)]
                we = egbuf2[p, i, pl.ds(jj * 16, 16)]
                za = _unlo(ws_a) + _unlo(wd) + _unlo(we)
                zb = _unlo(ws_b) + _unhi(wd) + _unhi(we)
                mbuf2[p, i, pl.ds(jj * 16, 16)] = (
                    _unhi(ws_a) / (1.0 + jnp.exp(-za)))
                mbuf2[p, i, pl.ds((jj + 4) * 16, 16)] = (
                    _unhi(ws_b) / (1.0 + jnp.exp(-zb)))
            wy = sbuf2[p, i, pl.ds(DIM, 16)]
            wyd = dbuf2[p, i, pl.ds(HDIM, 16)]
            s16buf[p * (HALF // PACK) + i // PACK,
                   pl.ds((i % PACK) * EDIM, EDIM)] = _unlo(wy) + _unlo(wyd)

        pltpu.sync_copy(mbuf2.at[p], aggsh.at[didx2.at[p]], add=True)

    rb0 = start * CHUNK
    pltpu.sync_copy(src_idx.at[pl.ds(rb0, RINGE)], sring.at[pl.ds(0, RINGE)])
    pltpu.sync_copy(dst_idx.at[pl.ds(rb0, RINGE)], dring.at[pl.ds(0, RINGE)])
    _fill(0, 0, 0)
    _issue(0, start, 0)

    def _chunk(t, carry):
        ci = start + t

        @pl.when(t % RING == 0)
        def _refill():
            w1 = t // RING + 1
            rb = (start + w1 * RING) * CHUNK
            ro = (w1 % 2) * RINGE
            pltpu.sync_copy(src_idx.at[pl.ds(rb, RINGE)],
                            sring.at[pl.ds(ro, RINGE)])
            pltpu.sync_copy(dst_idx.at[pl.ds(rb, RINGE)],
                            dring.at[pl.ds(ro, RINGE)])

        valid = t < nch

        @pl.when(valid)
        def _half_b_issue():
            _fill(1, t, 1)
            _issue(1, ci, 1)

        @pl.when(valid)
        def _half_a_run():
            _wait(0)
            _compute(0)

        @pl.when(t + 1 < nch)
        def _half_a_next():
            _fill(0, t + 1, 0)
            _issue(0, ci + 1, 0)

        @pl.when(valid)
        def _half_b_run():
            _wait(1)
            _compute(1)
            pltpu.sync_copy(s16buf,
                            s16_out.at[pl.ds(ci * (CHUNK // PACK),
                                             CHUNK // PACK)])

        return carry

    lax.fori_loop(0, MAXCH, _chunk, 0)

    plsc.subcore_barrier()
    pltpu.sync_copy(aggsh.at[pl.ds(rbase, RPT)],
                    agg_out.at[cid, pl.ds(rbase, RPT)])

    @pl.when(sid == 0)
    def _write_tail():
        pltpu.sync_copy(aggsh.at[pl.ds(NS * RPT, RTAIL)],
                        agg_out.at[cid, pl.ds(NS * RPT, RTAIL)])


def _node_post_body(h_ref, a0_ref, a1_ref, w1h_ref, w1a_ref, b1_ref,
                    w2_ref, b2_ref, g_ref, bn_ref, o_ref):
    h = h_ref[...]
    agg = a0_ref[0] + a1_ref[0]
    t = (jnp.dot(h, w1h_ref[...], preferred_element_type=_F32)
         + jnp.dot(agg, w1a_ref[...], preferred_element_type=_F32)
         + b1_ref[...])
    t = t * _sigmoid(t)
    nu = jnp.dot(t, w2_ref[...], preferred_element_type=_F32) + b2_ref[...]
    x = h + nu
    mu = jnp.mean(x, axis=1, keepdims=True)
    d = x - mu
    var = jnp.mean(d * d, axis=1, keepdims=True)
    o_ref[...] = d * lax.rsqrt(var + 1e-5) * g_ref[...] + bn_ref[...]


_node_post = pl.pallas_call(
    _node_post_body,
    grid=(NN // _BN,),
    in_specs=[
        pl.BlockSpec((_BN, DIM), lambda i: (i, 0)),
        pl.BlockSpec((1, _BN, DIM), lambda i: (0, i, 0)),
        pl.BlockSpec((1, _BN, DIM), lambda i: (1, i, 0)),
        pl.BlockSpec((DIM, DIM), lambda i: (0, 0)),
        pl.BlockSpec((DIM, DIM), lambda i: (0, 0)),
        pl.BlockSpec((1, DIM), lambda i: (0, 0)),
        pl.BlockSpec((DIM, DIM), lambda i: (0, 0)),
        pl.BlockSpec((1, DIM), lambda i: (0, 0)),
        pl.BlockSpec((1, DIM), lambda i: (0, 0)),
        pl.BlockSpec((1, DIM), lambda i: (0, 0)),
    ],
    out_specs=pl.BlockSpec((_BN, DIM), lambda i: (i, 0)),
    out_shape=jax.ShapeDtypeStruct((NN, DIM), _F32),
)


_BEP = 8000


def _edge_post_body(e_ref, s_ref, w1_ref, b1_ref, w2_ref, b2_ref,
                    gm_ref, g_ref, bn_ref, o_ref):
    eb = e_ref[...]
    t1 = (jnp.dot(eb, w1_ref[...], preferred_element_type=_F32)
          + s_ref[...] + b1_ref[...])
    t = t1 * _sigmoid(t1)
    eu = jnp.dot(t, w2_ref[...], preferred_element_type=_F32) + b2_ref[...]
    x = eb + eu
    mu = jnp.dot(x, gm_ref[...], preferred_element_type=_F32)
    d = x - mu
    var = jnp.dot(d * d, gm_ref[...], preferred_element_type=_F32)
    o_ref[...] = d * lax.rsqrt(var + 1e-5) * g_ref[...] + bn_ref[...]


_edge_post = pl.pallas_call(
    _edge_post_body,
    grid=(NER // _BEP,),
    in_specs=[
        pl.BlockSpec((_BEP, DIM), lambda i: (i, 0)),
        pl.BlockSpec((_BEP, DIM), lambda i: (i, 0)),
        pl.BlockSpec((DIM, DIM), lambda i: (0, 0)),
        pl.BlockSpec((1, DIM), lambda i: (0, 0)),
        pl.BlockSpec((DIM, DIM), lambda i: (0, 0)),
        pl.BlockSpec((1, DIM), lambda i: (0, 0)),
        pl.BlockSpec((DIM, DIM), lambda i: (0, 0)),
        pl.BlockSpec((1, DIM), lambda i: (0, 0)),
        pl.BlockSpec((1, DIM), lambda i: (0, 0)),
    ],
    out_specs=pl.BlockSpec((_BEP, DIM), lambda i: (i, 0)),
    out_shape=jax.ShapeDtypeStruct((NER, DIM), _F32),
)


def kernel(h, e, edge_index, params):
    p = params
    src = jnp.pad(edge_index[0].astype(jnp.int32), (0, IDXPAD))
    dst = jnp.pad(edge_index[1].astype(jnp.int32), (0, IDXPAD))

    wys = p['eu_W1'][:, EDIM:EDIM + DIM].T
    wyd = p['eu_W1'][:, EDIM + DIM:].T
    z48 = jnp.zeros((DIM, SRCW - DIM - EDIM), _F32)
    wls = jnp.concatenate([p['src_W'].T, p['msg_W'].T @ wys, z48], axis=1)
    bls = jnp.concatenate([p['src_b'], p['msg_b'] @ wys,
                           jnp.zeros((SRCW - DIM - EDIM,), _F32)])[None, :]
    whs = jnp.concatenate([p['msg_W'].T, jnp.zeros((DIM, SRCW - DIM), _F32)],
                          axis=1)
    bhs = jnp.concatenate([p['msg_b'],
                           jnp.zeros((SRCW - DIM,), _F32)])[None, :]
    zd = jnp.zeros((DIM, DSTW - HDIM - EDIM), _F32)
    wld = jnp.concatenate([p['dst_W'].T[:, :HDIM], p['msg_W'].T @ wyd, zd],
                          axis=1)
    bld = jnp.concatenate([p['dst_b'][:HDIM], p['msg_b'] @ wyd,
                           jnp.zeros((DSTW - HDIM - EDIM,), _F32)])[None, :]
    whd = jnp.concatenate([p['dst_W'].T[:, HDIM:],
                           jnp.zeros((DIM, DSTW - HDIM), _F32)], axis=1)
    bhd = jnp.concatenate([p['dst_b'][HDIM:],
                           jnp.zeros((DSTW - HDIM,), _F32)])[None, :]
    wle = p['eg_W'].T[:, :HDIM]
    ble = p['eg_b'][:HDIM][None, :]
    whe = p['eg_W'].T[:, HDIM:]
    bhe = p['eg_b'][HDIM:][None, :]

    eye8 = jnp.eye(PACK, dtype=_F32)
    w1bd = jnp.kron(eye8, p['eu_W1'][:, :EDIM].T)
    w2bd = jnp.kron(eye8, p['eu_W2'].T)
    gmat = jnp.kron(eye8, jnp.full((EDIM, EDIM), 1.0 / EDIM, _F32))
    b1t = jnp.tile(p['eu_b1'], PACK)[None, :]
    b2t = jnp.tile(p['eu_b2'], PACK)[None, :]
    egt = jnp.tile(p['en_g'], PACK)[None, :]
    ebt = jnp.tile(p['en_b'], PACK)[None, :]

    src_tab, dst_tab = _node_pre(h, wls, bls, whs, bhs, wld, bld, whd, bhd)
    eg = _eg_pre(e, wle, ble, whe, bhe)
    agg2, s16 = _sc_edge(src_tab, dst_tab, eg, src, dst)

    h_new = _node_post(h, agg2, agg2,
                       p['nu_W1'][:, :DIM].T, p['nu_W1'][:, DIM:].T,
                       p['nu_b1'][None, :], p['nu_W2'].T,
                       p['nu_b2'][None, :], p['nn_g'][None, :],
                       p['nn_b'][None, :])
    e_new = _edge_post(e.reshape(NER, DIM), s16,
                       w1bd, b1t, w2bd, b2t, gmat, egt, ebt)
    return (h_new, e_new.reshape(NE, EDIM))
